# baseline (speedup 1.0000x reference)
import functools

import jax
import jax.numpy as jnp
from jax.experimental import pallas as pl
from jax.experimental.pallas import tpu as pltpu

BF16 = jnp.bfloat16

_G = 8
_DM = 232
_PM = 208


def _image_kernel(s1_ref, s2_ref, w1_ref, b1_ref, w2_ref, b2_ref,
                  w3_ref, b3_ref, wf1_ref, bf1_ref, wf2_ref, bf2_ref,
                  wf3_ref, bf3_ref,
                  o_ref,
                  xt_buf, cat1_buf, c1_buf, pool_buf, pcl_buf, pc_buf,
                  cat2_buf, c2_buf, cat3_buf, c3_buf, catf_buf,
                  *, W, G, hw):
    f32 = jnp.float32
    NC = G * 512

    for g in range(G):
        x3 = jnp.concatenate(
            [s1_ref[8 * g:8 * g + 8], s2_ref[8 * g:8 * g + 8]], axis=1)
        xt_buf[g] = jnp.transpose(x3.reshape(64, hw), (1, 0))

        for t, sh in enumerate((0, 1, W, W + 1)):
            cat1_buf[g, :, 64 * t:64 * (t + 1)] = (
                xt_buf[g, sh:sh + _DM, :].astype(BF16))

        c1_buf[g] = jnp.maximum(
            jnp.dot(cat1_buf[g], w1_ref[...], preferred_element_type=f32)
            + b1_ref[...], 0.0)

        p = jnp.maximum(c1_buf[g, 0:_PM], c1_buf[g, 1:_PM + 1])
        p = jnp.maximum(p, c1_buf[g, W:_PM + W])
        pool_buf[g] = jnp.maximum(p, c1_buf[g, W + 1:_PM + W + 1])

        for u in range(7):
            pcl_buf[g, 8 * u:8 * u + 7, :] = (
                pool_buf[g, pl.ds(2 * u * W, 7, stride=2), :])

        for img in range(8):
            pc_buf[g * 512 + img * 64:g * 512 + (img + 1) * 64, :] = (
                pcl_buf[g, :, 16 * img:16 * (img + 1)])

    for t, sh in enumerate((0, 1, 8, 9)):
        cat2_buf[:, 16 * t:16 * (t + 1)] = pc_buf[sh:sh + NC, :].astype(BF16)
    c2_buf[0:NC, :] = jnp.maximum(
        jnp.dot(cat2_buf[...], w2_ref[...], preferred_element_type=f32)
        + b2_ref[...], 0.0)

    for t, sh in enumerate((0, 1, 8, 9)):
        cat3_buf[:, 32 * t:32 * (t + 1)] = c2_buf[sh:sh + NC, :].astype(BF16)
    c3_buf[...] = jnp.maximum(
        jnp.dot(cat3_buf[...], w3_ref[...], preferred_element_type=f32)
        + b3_ref[...], 0.0)

    for i in range(5):
        for j in range(5):
            k = i * 5 + j
            catf_buf[:, 64 * k:64 * (k + 1)] = (
                c3_buf[pl.ds(i * 8 + j, 8 * G, stride=64), :].astype(BF16))

    h = jnp.maximum(
        jnp.dot(catf_buf[...], wf1_ref[...], preferred_element_type=f32)
        + bf1_ref[...], 0.0)
    h = jnp.maximum(
        jnp.dot(h.astype(BF16), wf2_ref[...], preferred_element_type=f32)
        + bf2_ref[...], 0.0)
    o_ref[...] = (jnp.dot(h.astype(BF16), wf3_ref[...],
                          preferred_element_type=f32)
                  + bf3_ref[...]).astype(o_ref.dtype)


def kernel(w1, b1, w2, b2, w3, b3, wf1, bf1, wf2, bf2, wf3, bf3,
           state_1, state_2):
    C, H, W = state_1.shape[1], state_1.shape[2], state_1.shape[3]
    B = state_1.shape[0]
    HW = H * W
    action_size = wf3.shape[1]
    hidden_dim = wf1.shape[1]

    G = _G
    nb = 8 * G
    steps = -(-B // nb)
    B_pad = steps * nb
    NC = G * 512

    s1 = state_1 if B == B_pad else jnp.pad(
        state_1, ((0, B_pad - B), (0, 0), (0, 0), (0, 0)))
    s2 = state_2 if B == B_pad else jnp.pad(
        state_2, ((0, B_pad - B), (0, 0), (0, 0), (0, 0)))

    w1r = w1.astype(BF16).reshape(4, 2 * C, 16)
    eye = jnp.eye(8, dtype=BF16)
    w1bd = jnp.stack([jnp.kron(eye, w1r[t]) for t in range(4)], 0)
    w1bd = w1bd.reshape(4 * 8 * 2 * C, 8 * 16)
    b1l = jnp.tile(b1, (1, 8))

    body = functools.partial(_image_kernel, W=W, G=G, hw=HW)

    const = lambda shape: pl.BlockSpec(shape, lambda b: tuple(0 for _ in shape))
    in_specs = [
        pl.BlockSpec((nb, C, H, W), lambda b: (b, 0, 0, 0)),
        pl.BlockSpec((nb, C, H, W), lambda b: (b, 0, 0, 0)),
        const((4 * 8 * 2 * C, 8 * 16)), const((1, 8 * 16)),
        const((4 * 16, 32)), const((1, 32)),
        const((4 * 32, 64)), const((1, 64)),
        const((25 * 64, hidden_dim)), const((1, hidden_dim)),
        const((hidden_dim, hidden_dim)), const((1, hidden_dim)),
        const((hidden_dim, action_size)), const((1, action_size)),
    ]

    out = pl.pallas_call(
        body,
        out_shape=jax.ShapeDtypeStruct((steps, nb, action_size), jnp.float32),
        grid=(steps,),
        in_specs=in_specs,
        out_specs=pl.BlockSpec((None, nb, action_size), lambda b: (b, 0, 0)),
        scratch_shapes=[
            pltpu.VMEM((G, HW, 64), jnp.float32),
            pltpu.VMEM((G, _DM, 256), BF16),
            pltpu.VMEM((G, _DM, 128), jnp.float32),
            pltpu.VMEM((G, _PM, 128), jnp.float32),
            pltpu.VMEM((G, 64, 128), jnp.float32),
            pltpu.VMEM((NC + 16, 16), jnp.float32),
            pltpu.VMEM((NC, 4 * 16), BF16),
            pltpu.VMEM((NC + 16, 32), jnp.float32),
            pltpu.VMEM((NC, 4 * 32), BF16),
            pltpu.VMEM((NC, 64), jnp.float32),
            pltpu.VMEM((8 * G, 25 * 64), BF16),
        ],
        compiler_params=pltpu.CompilerParams(
            dimension_semantics=("parallel",)),
    )(s1, s2,
      w1bd, b1l, w2.astype(BF16), b2, w3.astype(BF16), b3,
      wf1.astype(BF16), bf1, wf2.astype(BF16), bf2, wf3.astype(BF16), bf3)
    return out.reshape(B_pad, action_size)[:B]


# speedup vs baseline: 1.4151x; 1.4151x over previous
import functools

import jax
import jax.numpy as jnp
from jax.experimental import pallas as pl
from jax.experimental.pallas import tpu as pltpu

BF16 = jnp.bfloat16

_G = 32
_DM = 232
_PM = 208


def _image_kernel(s1_ref, s2_ref, w1_ref, b1_ref, w2_ref, b2_ref,
                  w3_ref, b3_ref, wf1_ref, bf1_ref, wf2_ref, bf2_ref,
                  wf3_ref, bf3_ref,
                  o_ref,
                  xt_buf, cat1_buf, c1_buf, pool_buf, pcl_buf, pc_buf,
                  cat2_buf, c2_buf, cat3_buf, c3_buf, catf_buf,
                  *, W, G, hw):
    f32 = jnp.float32
    NC = G * 512

    for g in range(G):
        x3 = jnp.concatenate(
            [s1_ref[8 * g:8 * g + 8], s2_ref[8 * g:8 * g + 8]], axis=1)
        xt_buf[g] = jnp.transpose(x3.reshape(64, hw), (1, 0))

        for t, sh in enumerate((0, 1, W, W + 1)):
            cat1_buf[g, :, 64 * t:64 * (t + 1)] = (
                xt_buf[g, sh:sh + _DM, :].astype(BF16))

        c1_buf[g] = jnp.maximum(
            jnp.dot(cat1_buf[g], w1_ref[...], preferred_element_type=f32)
            + b1_ref[...], 0.0).astype(BF16)

        p = jnp.maximum(c1_buf[g, 0:_PM], c1_buf[g, 1:_PM + 1])
        p = jnp.maximum(p, c1_buf[g, W:_PM + W])
        pool_buf[g] = jnp.maximum(
            p, c1_buf[g, W + 1:_PM + W + 1]).astype(jnp.float32)

        for u in range(7):
            pcl_buf[g, 8 * u:8 * u + 7, :] = (
                pool_buf[g, pl.ds(2 * u * W, 7, stride=2), :].astype(BF16))

        for img in range(8):
            pc_buf[g * 512 + img * 64:g * 512 + (img + 1) * 64, :] = (
                pcl_buf[g, :, 16 * img:16 * (img + 1)])

    for t, sh in enumerate((0, 1, 8, 9)):
        cat2_buf[:, 16 * t:16 * (t + 1)] = pc_buf[sh:sh + NC, :]
    c2_buf[0:NC, :] = jnp.maximum(
        jnp.dot(cat2_buf[...], w2_ref[...], preferred_element_type=f32)
        + b2_ref[...], 0.0).astype(BF16)

    for t, sh in enumerate((0, 1, 8, 9)):
        cat3_buf[:, 32 * t:32 * (t + 1)] = c2_buf[sh:sh + NC, :]
    c3_buf[...] = jnp.maximum(
        jnp.dot(cat3_buf[...], w3_ref[...], preferred_element_type=f32)
        + b3_ref[...], 0.0)

    for i in range(5):
        for j in range(5):
            k = i * 5 + j
            catf_buf[:, 64 * k:64 * (k + 1)] = (
                c3_buf[pl.ds(i * 8 + j, 8 * G, stride=64), :].astype(BF16))

    h = jnp.maximum(
        jnp.dot(catf_buf[...], wf1_ref[...], preferred_element_type=f32)
        + bf1_ref[...], 0.0)
    h = jnp.maximum(
        jnp.dot(h.astype(BF16), wf2_ref[...], preferred_element_type=f32)
        + bf2_ref[...], 0.0)
    o_ref[...] = (jnp.dot(h.astype(BF16), wf3_ref[...],
                          preferred_element_type=f32)
                  + bf3_ref[...]).astype(o_ref.dtype)


def kernel(w1, b1, w2, b2, w3, b3, wf1, bf1, wf2, bf2, wf3, bf3,
           state_1, state_2):
    C, H, W = state_1.shape[1], state_1.shape[2], state_1.shape[3]
    B = state_1.shape[0]
    HW = H * W
    action_size = wf3.shape[1]
    hidden_dim = wf1.shape[1]

    G = _G
    nb = 8 * G
    steps = -(-B // nb)
    B_pad = steps * nb
    NC = G * 512

    s1 = state_1.reshape(steps, nb, C, HW) if B == B_pad else (
        jnp.pad(state_1.reshape(B, C, HW), ((0, B_pad - B), (0, 0), (0, 0)))
        .reshape(steps, nb, C, HW))
    s2 = state_2.reshape(steps, nb, C, HW) if B == B_pad else (
        jnp.pad(state_2.reshape(B, C, HW), ((0, B_pad - B), (0, 0), (0, 0)))
        .reshape(steps, nb, C, HW))

    w1r = w1.astype(BF16).reshape(4, 2 * C, 16)
    eye = jnp.eye(8, dtype=BF16)
    w1bd = jnp.stack([jnp.kron(eye, w1r[t]) for t in range(4)], 0)
    w1bd = w1bd.reshape(4 * 8 * 2 * C, 8 * 16)
    b1l = jnp.tile(b1, (1, 8))

    body = functools.partial(_image_kernel, W=W, G=G, hw=HW)

    const = lambda shape: pl.BlockSpec(shape, lambda b: tuple(0 for _ in shape))
    in_specs = [
        pl.BlockSpec((None, nb, C, HW), lambda b: (b, 0, 0, 0)),
        pl.BlockSpec((None, nb, C, HW), lambda b: (b, 0, 0, 0)),
        const((4 * 8 * 2 * C, 8 * 16)), const((1, 8 * 16)),
        const((4 * 16, 32)), const((1, 32)),
        const((4 * 32, 64)), const((1, 64)),
        const((25 * 64, hidden_dim)), const((1, hidden_dim)),
        const((hidden_dim, hidden_dim)), const((1, hidden_dim)),
        const((hidden_dim, action_size)), const((1, action_size)),
    ]

    out = pl.pallas_call(
        body,
        out_shape=jax.ShapeDtypeStruct((steps, nb, action_size), jnp.float32),
        grid=(steps,),
        in_specs=in_specs,
        out_specs=pl.BlockSpec((None, nb, action_size), lambda b: (b, 0, 0)),
        scratch_shapes=[
            pltpu.VMEM((G, HW, 64), jnp.float32),
            pltpu.VMEM((G, _DM, 256), BF16),
            pltpu.VMEM((G, _DM, 128), BF16),
            pltpu.VMEM((G, _PM, 128), jnp.float32),
            pltpu.VMEM((G, 64, 128), BF16),
            pltpu.VMEM((NC + 16, 16), BF16),
            pltpu.VMEM((NC, 4 * 16), BF16),
            pltpu.VMEM((NC + 16, 32), BF16),
            pltpu.VMEM((NC, 4 * 32), BF16),
            pltpu.VMEM((NC, 64), jnp.float32),
            pltpu.VMEM((8 * G, 25 * 64), BF16),
        ],
        compiler_params=pltpu.CompilerParams(
            dimension_semantics=("parallel",)),
    )(s1, s2,
      w1bd, b1l, w2.astype(BF16), b2, w3.astype(BF16), b3,
      wf1.astype(BF16), bf1, wf2.astype(BF16), bf2, wf3.astype(BF16), bf3)
    return out.reshape(B_pad, action_size)[:B]


# speedup vs baseline: 2.2413x; 1.5839x over previous
import functools

import jax
import jax.numpy as jnp
from jax.experimental import pallas as pl
from jax.experimental.pallas import tpu as pltpu

BF16 = jnp.bfloat16

_G = 32
_DM = 232
_PM = 208


def _image_kernel(s1_ref, s2_ref, w1_ref, b1_ref, w2_ref, b2_ref,
                  w3_ref, b3_ref, wf1_ref, bf1_ref, wf2_ref, bf2_ref,
                  wf3_ref, bf3_ref,
                  o_ref,
                  xt_buf, cat1_buf, c1_buf, pool_buf, pcl_buf,
                  cat2_buf, c2_buf, cat3_buf, c3r_buf, catf_buf,
                  *, W, G, hw):
    f32 = jnp.float32

    for g in range(G):
        x3 = jnp.concatenate(
            [s1_ref[8 * g:8 * g + 8], s2_ref[8 * g:8 * g + 8]], axis=1)
        xt_buf[g] = jnp.transpose(x3.reshape(64, hw), (1, 0))

        for t, sh in enumerate((0, 1, W, W + 1)):
            cat1_buf[g, :, 64 * t:64 * (t + 1)] = (
                xt_buf[g, sh:sh + _DM, :].astype(BF16))

        c1_buf[g] = jnp.maximum(
            jnp.dot(cat1_buf[g], w1_ref[...], preferred_element_type=f32)
            + b1_ref[...], 0.0)

        p = jnp.maximum(c1_buf[g, 0:_PM], c1_buf[g, 1:_PM + 1])
        p = jnp.maximum(p, c1_buf[g, W:_PM + W])
        pool_buf[g] = jnp.maximum(p, c1_buf[g, W + 1:_PM + W + 1])

        for u in range(7):
            pcl_buf[g, 8 * u:8 * u + 7, :] = (
                pool_buf[g, pl.ds(2 * u * W, 7, stride=2), :])

        for t, sh in enumerate((0, 1, 8, 9)):
            cat2_buf[g, :, 128 * t:128 * (t + 1)] = (
                pcl_buf[g, sh:sh + 64, :].astype(BF16))
        c2_buf[g, 0:64] = jnp.maximum(
            jnp.dot(cat2_buf[g], w2_ref[...], preferred_element_type=f32)
            + b2_ref[...], 0.0)

        for t, sh in enumerate((0, 1, 8, 9)):
            cat3_buf[g, :, 256 * t:256 * (t + 1)] = (
                c2_buf[g, sh:sh + 64, :].astype(BF16))
        c3l = jnp.maximum(
            jnp.dot(cat3_buf[g], w3_ref[...], preferred_element_type=f32)
            + b3_ref[...], 0.0)

        for img in range(8):
            c3r_buf[(8 * g + img) * 64:(8 * g + img + 1) * 64, :] = (
                c3l[:, 64 * img:64 * (img + 1)])

    for i in range(5):
        for j in range(5):
            k = i * 5 + j
            catf_buf[:, 64 * k:64 * (k + 1)] = (
                c3r_buf[pl.ds(i * 8 + j, 8 * G, stride=64), :].astype(BF16))

    h = jnp.maximum(
        jnp.dot(catf_buf[...], wf1_ref[...], preferred_element_type=f32)
        + bf1_ref[...], 0.0)
    h = jnp.maximum(
        jnp.dot(h.astype(BF16), wf2_ref[...], preferred_element_type=f32)
        + bf2_ref[...], 0.0)
    o_ref[...] = (jnp.dot(h.astype(BF16), wf3_ref[...],
                          preferred_element_type=f32)
                  + bf3_ref[...]).astype(o_ref.dtype)


def _blockdiag(w, taps, kpt, eye8):
    blocks = [jnp.kron(eye8, w[kpt * t:kpt * (t + 1), :]) for t in range(taps)]
    return jnp.concatenate(blocks, axis=0)


def kernel(w1, b1, w2, b2, w3, b3, wf1, bf1, wf2, bf2, wf3, bf3,
           state_1, state_2):
    C, H, W = state_1.shape[1], state_1.shape[2], state_1.shape[3]
    B = state_1.shape[0]
    HW = H * W
    action_size = wf3.shape[1]
    hidden_dim = wf1.shape[1]

    G = _G
    nb = 8 * G
    steps = -(-B // nb)
    B_pad = steps * nb

    s1 = state_1.reshape(steps, nb, C, HW) if B == B_pad else (
        jnp.pad(state_1.reshape(B, C, HW), ((0, B_pad - B), (0, 0), (0, 0)))
        .reshape(steps, nb, C, HW))
    s2 = state_2.reshape(steps, nb, C, HW) if B == B_pad else (
        jnp.pad(state_2.reshape(B, C, HW), ((0, B_pad - B), (0, 0), (0, 0)))
        .reshape(steps, nb, C, HW))

    eye8 = jnp.eye(8, dtype=BF16)
    w1bd = _blockdiag(w1.astype(BF16), 4, 2 * C, eye8)
    w2bd = _blockdiag(w2.astype(BF16), 4, 16, eye8)
    w3bd = _blockdiag(w3.astype(BF16), 4, 32, eye8)
    b1l = jnp.tile(b1, (1, 8))
    b2l = jnp.tile(b2, (1, 8))
    b3l = jnp.tile(b3, (1, 8))

    body = functools.partial(_image_kernel, W=W, G=G, hw=HW)

    const = lambda shape: pl.BlockSpec(shape, lambda b: tuple(0 for _ in shape))
    in_specs = [
        pl.BlockSpec((None, nb, C, HW), lambda b: (b, 0, 0, 0)),
        pl.BlockSpec((None, nb, C, HW), lambda b: (b, 0, 0, 0)),
        const((4 * 8 * 2 * C, 8 * 16)), const((1, 8 * 16)),
        const((4 * 8 * 16, 8 * 32)), const((1, 8 * 32)),
        const((4 * 8 * 32, 8 * 64)), const((1, 8 * 64)),
        const((25 * 64, hidden_dim)), const((1, hidden_dim)),
        const((hidden_dim, hidden_dim)), const((1, hidden_dim)),
        const((hidden_dim, action_size)), const((1, action_size)),
    ]

    out = pl.pallas_call(
        body,
        out_shape=jax.ShapeDtypeStruct((steps, nb, action_size), jnp.float32),
        grid=(steps,),
        in_specs=in_specs,
        out_specs=pl.BlockSpec((None, nb, action_size), lambda b: (b, 0, 0)),
        scratch_shapes=[
            pltpu.VMEM((G, HW, 64), jnp.float32),
            pltpu.VMEM((G, _DM, 256), BF16),
            pltpu.VMEM((G, _DM, 128), jnp.float32),
            pltpu.VMEM((G, _PM, 128), jnp.float32),
            pltpu.VMEM((G, 80, 128), jnp.float32),
            pltpu.VMEM((G, 64, 512), BF16),
            pltpu.VMEM((G, 80, 256), jnp.float32),
            pltpu.VMEM((G, 64, 1024), BF16),
            pltpu.VMEM((8 * G * 64, 64), jnp.float32),
            pltpu.VMEM((8 * G, 25 * 64), BF16),
        ],
        compiler_params=pltpu.CompilerParams(
            dimension_semantics=("parallel",)),
    )(s1, s2,
      w1bd, b1l, w2bd, b2l, w3bd, b3l,
      wf1.astype(BF16), bf1, wf2.astype(BF16), bf2, wf3.astype(BF16), bf3)
    return out.reshape(B_pad, action_size)[:B]


# speedup vs baseline: 2.2648x; 1.0105x over previous
import functools

import jax
import jax.numpy as jnp
from jax.experimental import pallas as pl
from jax.experimental.pallas import tpu as pltpu

BF16 = jnp.bfloat16

_G = 32
_DM = 232
_PM = 208


def _image_kernel(s1_ref, s2_ref, w1_ref, b1_ref, w2_ref, b2_ref,
                  w3_ref, b3_ref, wf1_ref, bf1_ref, wf2_ref, bf2_ref,
                  wf3_ref, bf3_ref,
                  o_ref,
                  xt_buf, cat1_buf, c1_buf, pool_buf, pcl_buf,
                  cat2_buf, c2_buf, cat3_buf, c3r_buf, catf_buf,
                  *, W, G, hw):
    f32 = jnp.float32

    for g in range(G):
        x3 = jnp.concatenate(
            [s1_ref[8 * g:8 * g + 8], s2_ref[8 * g:8 * g + 8]], axis=1)
        xt_buf[g] = jnp.transpose(x3.reshape(64, hw).astype(BF16), (1, 0))

        for t, sh in enumerate((0, 1, W, W + 1)):
            cat1_buf[g, :, 64 * t:64 * (t + 1)] = xt_buf[g, sh:sh + _DM, :]

        c1_buf[g] = jnp.maximum(
            jnp.dot(cat1_buf[g], w1_ref[...], preferred_element_type=f32)
            + b1_ref[...], 0.0)

        p = jnp.maximum(c1_buf[g, 0:_PM], c1_buf[g, 1:_PM + 1])
        p = jnp.maximum(p, c1_buf[g, W:_PM + W])
        pool_buf[g] = jnp.maximum(p, c1_buf[g, W + 1:_PM + W + 1])

        for u in range(7):
            pcl_buf[g, 8 * u:8 * u + 7, :] = (
                pool_buf[g, pl.ds(2 * u * W, 7, stride=2), :])

        for t, sh in enumerate((0, 1, 8, 9)):
            cat2_buf[g, :, 128 * t:128 * (t + 1)] = (
                pcl_buf[g, sh:sh + 64, :].astype(BF16))
        c2_buf[g, 0:64] = jnp.maximum(
            jnp.dot(cat2_buf[g], w2_ref[...], preferred_element_type=f32)
            + b2_ref[...], 0.0).astype(BF16)

        for t, sh in enumerate((0, 1, 8, 9)):
            cat3_buf[g, :, 256 * t:256 * (t + 1)] = c2_buf[g, sh:sh + 64, :]
        c3l = jnp.maximum(
            jnp.dot(cat3_buf[g], w3_ref[...], preferred_element_type=f32)
            + b3_ref[...], 0.0)

        for img in range(8):
            c3r_buf[(8 * g + img) * 64:(8 * g + img + 1) * 64, :] = (
                c3l[:, 64 * img:64 * (img + 1)])

    for i in range(5):
        for j in range(5):
            k = i * 5 + j
            catf_buf[:, 64 * k:64 * (k + 1)] = (
                c3r_buf[pl.ds(i * 8 + j, 8 * G, stride=64), :].astype(BF16))

    h = jnp.maximum(
        jnp.dot(catf_buf[...], wf1_ref[...], preferred_element_type=f32)
        + bf1_ref[...], 0.0)
    h = jnp.maximum(
        jnp.dot(h.astype(BF16), wf2_ref[...], preferred_element_type=f32)
        + bf2_ref[...], 0.0)
    o_ref[...] = (jnp.dot(h.astype(BF16), wf3_ref[...],
                          preferred_element_type=f32)
                  + bf3_ref[...]).astype(o_ref.dtype)


def _blockdiag(w, taps, kpt, eye8):
    blocks = [jnp.kron(eye8, w[kpt * t:kpt * (t + 1), :]) for t in range(taps)]
    return jnp.concatenate(blocks, axis=0)


def kernel(w1, b1, w2, b2, w3, b3, wf1, bf1, wf2, bf2, wf3, bf3,
           state_1, state_2):
    C, H, W = state_1.shape[1], state_1.shape[2], state_1.shape[3]
    B = state_1.shape[0]
    HW = H * W
    action_size = wf3.shape[1]
    hidden_dim = wf1.shape[1]

    G = _G
    nb = 8 * G
    steps = -(-B // nb)
    B_pad = steps * nb

    s1 = state_1.reshape(steps, nb, C, HW) if B == B_pad else (
        jnp.pad(state_1.reshape(B, C, HW), ((0, B_pad - B), (0, 0), (0, 0)))
        .reshape(steps, nb, C, HW))
    s2 = state_2.reshape(steps, nb, C, HW) if B == B_pad else (
        jnp.pad(state_2.reshape(B, C, HW), ((0, B_pad - B), (0, 0), (0, 0)))
        .reshape(steps, nb, C, HW))

    eye8 = jnp.eye(8, dtype=BF16)
    w1bd = _blockdiag(w1.astype(BF16), 4, 2 * C, eye8)
    w2bd = _blockdiag(w2.astype(BF16), 4, 16, eye8)
    w3bd = _blockdiag(w3.astype(BF16), 4, 32, eye8)
    b1l = jnp.tile(b1, (1, 8))
    b2l = jnp.tile(b2, (1, 8))
    b3l = jnp.tile(b3, (1, 8))

    body = functools.partial(_image_kernel, W=W, G=G, hw=HW)

    const = lambda shape: pl.BlockSpec(shape, lambda b: tuple(0 for _ in shape))
    in_specs = [
        pl.BlockSpec((None, nb, C, HW), lambda b: (b, 0, 0, 0)),
        pl.BlockSpec((None, nb, C, HW), lambda b: (b, 0, 0, 0)),
        const((4 * 8 * 2 * C, 8 * 16)), const((1, 8 * 16)),
        const((4 * 8 * 16, 8 * 32)), const((1, 8 * 32)),
        const((4 * 8 * 32, 8 * 64)), const((1, 8 * 64)),
        const((25 * 64, hidden_dim)), const((1, hidden_dim)),
        const((hidden_dim, hidden_dim)), const((1, hidden_dim)),
        const((hidden_dim, action_size)), const((1, action_size)),
    ]

    out = pl.pallas_call(
        body,
        out_shape=jax.ShapeDtypeStruct((steps, nb, action_size), jnp.float32),
        grid=(steps,),
        in_specs=in_specs,
        out_specs=pl.BlockSpec((None, nb, action_size), lambda b: (b, 0, 0)),
        scratch_shapes=[
            pltpu.VMEM((G, HW, 64), BF16),
            pltpu.VMEM((G, _DM, 256), BF16),
            pltpu.VMEM((G, _DM, 128), jnp.float32),
            pltpu.VMEM((G, _PM, 128), jnp.float32),
            pltpu.VMEM((G, 80, 128), jnp.float32),
            pltpu.VMEM((G, 64, 512), BF16),
            pltpu.VMEM((G, 80, 256), BF16),
            pltpu.VMEM((G, 64, 1024), BF16),
            pltpu.VMEM((8 * G * 64, 64), jnp.float32),
            pltpu.VMEM((8 * G, 25 * 64), BF16),
        ],
        compiler_params=pltpu.CompilerParams(
            dimension_semantics=("parallel",)),
    )(s1, s2,
      w1bd, b1l, w2bd, b2l, w3bd, b3l,
      wf1.astype(BF16), bf1, wf2.astype(BF16), bf2, wf3.astype(BF16), bf3)
    return out.reshape(B_pad, action_size)[:B]
